# baseline (speedup 1.0000x reference)
import functools

import jax
import jax.numpy as jnp
from jax import lax
from jax.experimental import pallas as pl
from jax.experimental.pallas import tpu as pltpu
from jax.experimental.pallas import tpu_sc as plsc

K = 64
V = 1000000
B = 16384
BS = 14
H = 50
NB = B * H

BJ = 8192
BJS = BJ.bit_length() - 1
GJ = -(-V // BJ)
VPAD = GJ * BJ


def _norm_t_body(p_ref, t_ref):
    x = p_ref[...]
    s = jnp.sum(x, axis=0, keepdims=True)
    y = (x / s).T
    t_ref[...] = jnp.concatenate([y[: BJ // 2], y[BJ // 2:]], axis=1)


def _normalize_transpose(probs):
    return pl.pallas_call(
        _norm_t_body,
        grid=(GJ,),
        in_specs=[pl.BlockSpec((K, BJ), lambda j: (j * 0, j))],
        out_specs=pl.BlockSpec((BJ // 2, 2 * K), lambda j: (j, j * 0)),
        out_shape=jax.ShapeDtypeStruct((VPAD // 2, 2 * K), jnp.float32),
    )(probs)


NC, NS = 2, 16
NW = NC * NS
PER_W = NB // NW
CHUNK = 512
NCHUNK = PER_W // CHUNK
L = 16

_sc_mesh = plsc.VectorSubcoreMesh(core_axis_name="c", subcore_axis_name="s")


@functools.partial(
    pl.kernel,
    mesh=_sc_mesh,
    out_type=jax.ShapeDtypeStruct((NB // 2, 2, K), jnp.float32),
    scratch_types=[
        pltpu.VMEM((NCHUNK, CHUNK), jnp.int32),
        pltpu.VMEM((CHUNK, K), jnp.float32),
        pltpu.VMEM((CHUNK, K), jnp.float32),
        pltpu.SemaphoreType.DMA,
        pltpu.SemaphoreType.DMA,
    ],
    compiler_params=pltpu.CompilerParams(use_tc_tiling_on_sc=False),
)
def _gather_rows(idx_hbm, table_hbm, out_hbm, idx_v, rows0, rows1, sem0, sem1):
    c32 = jnp.int32
    wid = lax.axis_index("s") * c32(NC) + lax.axis_index("c")
    base = wid * c32(PER_W)
    pltpu.sync_copy(idx_hbm.at[wid], idx_v)

    def remap(_, jj):
        for u in range(CHUNK // L):
            v = idx_v[jj, pl.ds(u * L, L)]
            l = v & c32(BJ - 1)
            idx_v[jj, pl.ds(u * L, L)] = (
                (v - l) + ((l & c32(BJ // 2 - 1)) << 1) + (l >> (BJS - 1))
            )
        return jj + c32(1)

    lax.fori_loop(0, NCHUNK, remap, c32(0))

    def start(jj, buf, sem):
        return pltpu.async_copy(table_hbm.at[idx_v.at[jj]], buf, sem)

    def drain(buf, sem):
        pltpu.make_async_copy(table_hbm.at[idx_v.at[c32(0)]], buf, sem).wait()

    def scat(buf, p):
        major = ((p >> c32(BS + 1)) << c32(BS)) + (p & c32(B - 1))
        c = (p >> c32(BS)) & c32(1)
        pltpu.sync_copy(
            buf, out_hbm.at[pl.ds(pl.multiple_of(major, CHUNK), CHUNK), c])

    start(c32(0), rows0, sem0)

    def body(_, carry):
        j, p = carry
        start(j + c32(1), rows1, sem1)
        drain(rows0, sem0)
        scat(rows0, p)

        @pl.when(j + c32(2) < c32(NCHUNK))
        def _():
            start(j + c32(2), rows0, sem0)

        drain(rows1, sem1)
        scat(rows1, p + c32(CHUNK))
        return (j + c32(2), p + c32(2 * CHUNK))

    lax.fori_loop(0, NCHUNK // 2, body, (c32(0), base))


BB = 2048


def _repack_body(x_ref, a_ref):
    x = x_ref[0]
    t = x.T
    a_ref[0] = t[:K]
    a_ref[1] = t[K:]


def _repack(x):
    return pl.pallas_call(
        _repack_body,
        grid=(H // 2, B // BB),
        in_specs=[pl.BlockSpec((1, BB, 2 * K), lambda m, b: (m, b, m * 0))],
        out_specs=pl.BlockSpec((2, K, BB), lambda m, b: (m, m * 0, b)),
        out_shape=jax.ShapeDtypeStruct((H, K, B), jnp.float32),
    )(x)


def kernel(data, probabilities):
    table = _normalize_transpose(probabilities).reshape(VPAD, K)
    idx = data.astype(jnp.int32).T.reshape(NW, NCHUNK, CHUNK)
    out = _gather_rows(idx, table)
    a = _repack(out.reshape(H // 2, B, 2 * K))
    return a.transpose(2, 0, 1)


# speedup vs baseline: 1.1054x; 1.1054x over previous
import functools

import jax
import jax.numpy as jnp
from jax import lax
from jax.experimental import pallas as pl
from jax.experimental.pallas import tpu as pltpu
from jax.experimental.pallas import tpu_sc as plsc

K = 64
V = 1000000
B = 16384
BS = 14
H = 50
NB = B * H

BJ = 8192
BJS = BJ.bit_length() - 1
GJ = -(-V // BJ)
VPAD = GJ * BJ


def _norm_t_body(p_ref, t_ref):
    x = p_ref[...]
    s = jnp.sum(x, axis=0, keepdims=True)
    y = (x / s).T
    t_ref[...] = jnp.concatenate([y[: BJ // 2], y[BJ // 2:]], axis=1)


def _normalize_transpose(probs):
    return pl.pallas_call(
        _norm_t_body,
        grid=(GJ,),
        in_specs=[pl.BlockSpec((K, BJ), lambda j: (j * 0, j))],
        out_specs=pl.BlockSpec((BJ // 2, 2 * K), lambda j: (j, j * 0)),
        out_shape=jax.ShapeDtypeStruct((VPAD // 2, 2 * K), jnp.float32),
    )(probs)


NC, NS = 2, 16
NW = NC * NS
PER_W = NB // NW
CHUNK = 512
NCHUNK = PER_W // CHUNK
L = 16

_sc_mesh = plsc.VectorSubcoreMesh(core_axis_name="c", subcore_axis_name="s")


@functools.partial(
    pl.kernel,
    mesh=_sc_mesh,
    out_type=jax.ShapeDtypeStruct((NB // 2, 2, K), jnp.float32),
    scratch_types=[
        pltpu.VMEM((NCHUNK, CHUNK), jnp.int32),
        pltpu.VMEM((CHUNK, K), jnp.float32),
        pltpu.VMEM((CHUNK, K), jnp.float32),
        pltpu.SemaphoreType.DMA,
        pltpu.SemaphoreType.DMA,
    ],
    compiler_params=pltpu.CompilerParams(use_tc_tiling_on_sc=False),
)
def _gather_rows(idx_hbm, table_hbm, out_hbm, idx_v, rows0, rows1, sem0, sem1):
    c32 = jnp.int32
    wid = lax.axis_index("s") * c32(NC) + lax.axis_index("c")
    base = wid * c32(PER_W)
    pltpu.sync_copy(idx_hbm.at[wid], idx_v)

    def remap(_, jj):
        for u in range(CHUNK // L):
            v = idx_v[jj, pl.ds(u * L, L)]
            l = v & c32(BJ - 1)
            idx_v[jj, pl.ds(u * L, L)] = (
                (v - l) + ((l & c32(BJ // 2 - 1)) << 1) + (l >> (BJS - 1))
            )
        return jj + c32(1)

    lax.fori_loop(0, NCHUNK, remap, c32(0))

    def start(jj, buf, sem):
        return pltpu.async_copy(table_hbm.at[idx_v.at[jj]], buf, sem)

    def drain(buf, sem):
        pltpu.make_async_copy(table_hbm.at[idx_v.at[c32(0)]], buf, sem).wait()

    def scat(buf, p):
        major = ((p >> c32(BS + 1)) << c32(BS)) + (p & c32(B - 1))
        c = (p >> c32(BS)) & c32(1)
        pltpu.sync_copy(
            buf, out_hbm.at[pl.ds(pl.multiple_of(major, CHUNK), CHUNK), c])

    start(c32(0), rows0, sem0)

    def body(_, carry):
        j, p = carry
        start(j + c32(1), rows1, sem1)
        drain(rows0, sem0)
        scat(rows0, p)

        @pl.when(j + c32(2) < c32(NCHUNK))
        def _():
            start(j + c32(2), rows0, sem0)

        drain(rows1, sem1)
        scat(rows1, p + c32(CHUNK))
        return (j + c32(2), p + c32(2 * CHUNK))

    lax.fori_loop(0, NCHUNK // 2, body, (c32(0), base))


BB = 4096


def _repack_body(x_ref, a_ref):
    x = x_ref[0]
    t = x.T
    a_ref[0] = t[:K]
    a_ref[1] = t[K:]


def _repack(x):
    return pl.pallas_call(
        _repack_body,
        grid=(H // 2, B // BB),
        in_specs=[pl.BlockSpec((1, BB, 2 * K), lambda m, b: (m, b, m * 0))],
        out_specs=pl.BlockSpec((2, K, BB), lambda m, b: (m, m * 0, b)),
        out_shape=jax.ShapeDtypeStruct((H, K, B), jnp.float32),
    )(x)


def kernel(data, probabilities):
    table = _normalize_transpose(probabilities).reshape(VPAD, K)
    idx = data.astype(jnp.int32).T.reshape(NW, NCHUNK, CHUNK)
    out = _gather_rows(idx, table)
    a = _repack(out.reshape(H // 2, B, 2 * K))
    return a.transpose(2, 0, 1)


# speedup vs baseline: 1.1517x; 1.0419x over previous
import functools

import jax
import jax.numpy as jnp
from jax import lax
from jax.experimental import pallas as pl
from jax.experimental.pallas import tpu as pltpu
from jax.experimental.pallas import tpu_sc as plsc

K = 64
V = 1000000
B = 16384
BS = 14
H = 50
NB = B * H

BJ = 8192
BJS = BJ.bit_length() - 1
GJ = -(-V // BJ)
VPAD = GJ * BJ


def _norm_t_body(p_ref, t_ref):
    x = p_ref[...]
    s = jnp.sum(x, axis=0, keepdims=True)
    y = (x / s).T
    t_ref[...] = jnp.concatenate([y[: BJ // 2], y[BJ // 2:]], axis=1)


def _normalize_transpose(probs):
    return pl.pallas_call(
        _norm_t_body,
        grid=(GJ,),
        in_specs=[pl.BlockSpec((K, BJ), lambda j: (j * 0, j))],
        out_specs=pl.BlockSpec((BJ // 2, 2 * K), lambda j: (j, j * 0)),
        out_shape=jax.ShapeDtypeStruct((VPAD // 2, 2 * K), jnp.float32),
    )(probs)


NC, NS = 2, 16
NW = NC * NS
PER_W = NB // NW
CHUNK = 512
NCHUNK = PER_W // CHUNK
L = 16

_sc_mesh = plsc.VectorSubcoreMesh(core_axis_name="c", subcore_axis_name="s")


@functools.partial(
    pl.kernel,
    mesh=_sc_mesh,
    out_type=jax.ShapeDtypeStruct((NB // 2, 2, K), jnp.float32),
    scratch_types=[
        pltpu.VMEM((NCHUNK, CHUNK), jnp.int32),
        pltpu.VMEM((CHUNK, K), jnp.float32),
        pltpu.VMEM((CHUNK, K), jnp.float32),
        pltpu.SemaphoreType.DMA,
        pltpu.SemaphoreType.DMA,
    ],
    compiler_params=pltpu.CompilerParams(use_tc_tiling_on_sc=False),
)
def _gather_rows(idx_hbm, table_hbm, out_hbm, idx_v, rows0, rows1, sem0, sem1):
    c32 = jnp.int32
    wid = lax.axis_index("s") * c32(NC) + lax.axis_index("c")
    base = wid * c32(PER_W)
    pltpu.sync_copy(idx_hbm.at[wid], idx_v)

    def remap(_, jj):
        for u in range(CHUNK // L):
            v = idx_v[jj, pl.ds(u * L, L)]
            l = v & c32(BJ - 1)
            idx_v[jj, pl.ds(u * L, L)] = (
                (v - l) + ((l & c32(BJ // 2 - 1)) << 1) + (l >> (BJS - 1))
            )
        return jj + c32(1)

    lax.fori_loop(0, NCHUNK, remap, c32(0))

    def start(jj, buf, sem):
        return pltpu.async_copy(table_hbm.at[idx_v.at[jj]], buf, sem)

    def drain(buf, sem):
        pltpu.make_async_copy(table_hbm.at[idx_v.at[c32(0)]], buf, sem).wait()

    def scat(buf, p):
        major = ((p >> c32(BS + 1)) << c32(BS)) + (p & c32(B - 1))
        c = (p >> c32(BS)) & c32(1)
        pltpu.sync_copy(
            buf, out_hbm.at[pl.ds(pl.multiple_of(major, CHUNK), CHUNK), c])

    start(c32(0), rows0, sem0)

    def body(_, carry):
        j, p = carry
        start(j + c32(1), rows1, sem1)
        drain(rows0, sem0)
        scat(rows0, p)

        @pl.when(j + c32(2) < c32(NCHUNK))
        def _():
            start(j + c32(2), rows0, sem0)

        drain(rows1, sem1)
        scat(rows1, p + c32(CHUNK))
        return (j + c32(2), p + c32(2 * CHUNK))

    lax.fori_loop(0, NCHUNK // 2, body, (c32(0), base))


BB = 8192


def _repack_body(x_ref, a_ref):
    x = x_ref[0]
    t = x.T
    a_ref[0] = t[:K]
    a_ref[1] = t[K:]


def _repack(x):
    return pl.pallas_call(
        _repack_body,
        grid=(H // 2, B // BB),
        in_specs=[pl.BlockSpec((1, BB, 2 * K), lambda m, b: (m, b, m * 0))],
        out_specs=pl.BlockSpec((2, K, BB), lambda m, b: (m, m * 0, b)),
        out_shape=jax.ShapeDtypeStruct((H, K, B), jnp.float32),
    )(x)


def kernel(data, probabilities):
    table = _normalize_transpose(probabilities).reshape(VPAD, K)
    idx = data.astype(jnp.int32).T.reshape(NW, NCHUNK, CHUNK)
    out = _gather_rows(idx, table)
    a = _repack(out.reshape(H // 2, B, 2 * K))
    return a.transpose(2, 0, 1)
